# baseline (speedup 1.0000x reference)
import jax
import jax.numpy as jnp
from jax import lax
from jax.experimental import pallas as pl
from jax.experimental.pallas import tpu as pltpu
from jax.experimental.pallas import tpu_sc as plsc

N_P = 100000
D_LAT = 128
HIDDEN = 64
D_DYN = 32

NC = 2
NS = 16
NW = NC * NS

CH = 40
NCH = N_P // CH
RAW = 3 * CH
SUB = 120
NSUB = RAW // SUB
RAW_BUF = ((RAW + 15) // 16) * 16
K2MAX = ((NCH + NW - 1) // NW + 2 + 1) // 2


def _k1_body(z_ref, wab_ref, wc_ref, uu_ref):
    z = z_ref[...]
    uu_ref[0] = jnp.dot(z, wab_ref[...], preferred_element_type=jnp.float32)
    uu_ref[1] = jnp.dot(z, wc_ref[...], preferred_element_type=jnp.float32)


def _k3_body(z_ref, s_ref, a_ref, b1_ref, w2_ref, b2_ref, out_ref):
    z = z_ref[...]
    h = jnp.maximum(
        jnp.dot(z, a_ref[...], preferred_element_type=jnp.float32)
        + s_ref[...] + b1_ref[...], 0.0)
    out_ref[...] = z + jnp.dot(h, w2_ref[...],
                               preferred_element_type=jnp.float32) + b2_ref[...]


def _make_sc_body(T0, T1, K2):
  def _sc_body(uu_hbm, nlf_hbm, s_hbm,
               raw0, raw1, g0, g1, s0, s1, semg0, semg1, semw0, semw1):
    wid = lax.axis_index("s") * NC + lax.axis_index("c")
    iota = lax.iota(jnp.int32, 16)
    patt = [(((iota + r) % 3) >> 1) * N_P for r in range(3)]
    raws, gs, ss = (raw0, raw1), (g0, g1), (s0, s1)
    semgs, semws = (semg0, semg1), (semw0, semw1)

    def zrow(i, c):
        for j in range(4):
            s0[i, pl.ds(64 + 16 * j, 16)] = jnp.zeros((16,), jnp.float32)
            s1[i, pl.ds(64 + 16 * j, 16)] = jnp.zeros((16,), jnp.float32)
        return c

    lax.fori_loop(0, CH, zrow, 0)

    def prep_and_fire(b, t):
        pltpu.sync_copy(nlf_hbm.at[pl.ds(t * RAW, RAW)], raws[b].at[pl.ds(0, RAW)])
        for q in range(RAW_BUF // 16):
            sl = pl.ds(16 * q, 16)
            raws[b][sl] = raws[b][sl] + patt[q % 3]
        for j in range(NSUB):
            pltpu.async_copy(
                uu_hbm.at[raws[b].at[pl.ds(j * SUB, SUB)]],
                gs[b].at[pl.ds(j * SUB, SUB)],
                semgs[b],
            )

    def wait_gathers(b, t):
        for j in range(NSUB):
            pltpu.make_async_copy(
                uu_hbm.at[raws[b].at[pl.ds(j * SUB, SUB)]],
                gs[b].at[pl.ds(j * SUB, SUB)],
                semgs[b],
            ).wait()

    def step(m, b):
        t = T0 + wid + m * NW
        tn = t + NW
        tp = t - 2 * NW

        @pl.when(tn < T1)
        def _():
            prep_and_fire(b ^ 1, tn)

        @pl.when(jnp.logical_and(tp >= T0, tp < T1))
        def _():
            pltpu.make_async_copy(
                ss[b], s_hbm.at[pl.ds((tp - T0) * CH, CH)], semws[b]).wait()

        @pl.when(t < T1)
        def _():
            wait_gathers(b, t)

            def red(i, c):
                for j in range(4):
                    sl = pl.ds(16 * j, 16)
                    ss[b][i, sl] = (gs[b][3 * i, sl]
                                    + gs[b][3 * i + 1, pl.ds(64 + 16 * j, 16)]
                                    + gs[b][3 * i + 2, sl])
                return c

            lax.fori_loop(0, CH, red, 0)
            pltpu.async_copy(ss[b], s_hbm.at[pl.ds((t - T0) * CH, CH)], semws[b])

    @pl.when(T0 + wid < T1)
    def _():
        prep_and_fire(0, T0 + wid)

    def pair(k2, carry):
        step(2 * k2, 0)
        step(2 * k2 + 1, 1)
        return carry

    lax.fori_loop(0, K2, pair, 0)

  return _sc_body


def kernel(z_old, neighbour_list, W1, b1, W2, b2):
    n = z_old.shape[0]
    assert n == N_P
    nlf = neighbour_list.astype(jnp.int32).reshape(-1)

    w1r = W1.reshape(D_LAT, 4, HIDDEN)
    zero_h = jnp.zeros((D_LAT, HIDDEN), jnp.float32)
    wab = jnp.concatenate([w1r[:, 1, :], w1r[:, 2, :]], axis=1)
    wc = jnp.concatenate([w1r[:, 3, :], zero_h], axis=1)
    a128 = jnp.concatenate([w1r[:, 0, :], zero_h], axis=1)
    w2ext = jnp.zeros((D_LAT, D_LAT), jnp.float32).at[:HIDDEN, :D_DYN].set(W2)
    b1pad = jnp.zeros((1, D_LAT), jnp.float32).at[0, :HIDDEN].set(b1)
    b2pad = jnp.zeros((1, D_LAT), jnp.float32).at[0, :D_DYN].set(b2)

    bn = 2000
    grid1 = (n // bn,)
    uu3 = pl.pallas_call(
        _k1_body,
        grid=grid1,
        in_specs=[
            pl.BlockSpec((bn, D_LAT), lambda i: (i, 0)),
            pl.BlockSpec((D_LAT, D_LAT), lambda i: (0, 0)),
            pl.BlockSpec((D_LAT, D_LAT), lambda i: (0, 0)),
        ],
        out_specs=pl.BlockSpec((2, bn, D_LAT), lambda i: (0, i, 0)),
        out_shape=jax.ShapeDtypeStruct((2, n, D_LAT), jnp.float32),
    )(z_old, wab, wc)
    uu = uu3.reshape(2 * n, D_LAT)

    sc_mesh = plsc.VectorSubcoreMesh(core_axis_name="c", subcore_axis_name="s")
    sc_scratch = [
        pltpu.VMEM((RAW_BUF,), jnp.int32),
        pltpu.VMEM((RAW_BUF,), jnp.int32),
        pltpu.VMEM((RAW, D_LAT), jnp.float32),
        pltpu.VMEM((RAW, D_LAT), jnp.float32),
        pltpu.VMEM((CH, D_LAT), jnp.float32),
        pltpu.VMEM((CH, D_LAT), jnp.float32),
        pltpu.SemaphoreType.DMA,
        pltpu.SemaphoreType.DMA,
        pltpu.SemaphoreType.DMA,
        pltpu.SemaphoreType.DMA,
    ]
    NSPLIT = 4
    HCH = NCH // NSPLIT
    K2H = ((HCH + NW - 1) // NW + 2 + 1) // 2
    s_part = []
    for h in range(NSPLIT):
        s_part.append(pl.kernel(
            _make_sc_body(h * HCH, (h + 1) * HCH, K2H),
            out_type=jax.ShapeDtypeStruct((n // NSPLIT, D_LAT), jnp.float32),
            mesh=sc_mesh,
            scratch_types=sc_scratch,
            compiler_params=pltpu.CompilerParams(use_tc_tiling_on_sc=False),
        )(uu, nlf))

    bnk = 5000
    nb_h = n // NSPLIT // bnk
    k3_common = dict(
        grid=(nb_h,),
        out_shape=jax.ShapeDtypeStruct((n, D_LAT), jnp.float32),
    )
    w_specs = [
        pl.BlockSpec((D_LAT, D_LAT), lambda i: (0, 0)),
        pl.BlockSpec((1, D_LAT), lambda i: (0, 0)),
        pl.BlockSpec((D_LAT, D_LAT), lambda i: (0, 0)),
        pl.BlockSpec((1, D_LAT), lambda i: (0, 0)),
    ]

    def _k3b_body(z_ref, s_ref, a_ref, b1_ref, w2_ref, b2_ref, prev_ref, out_ref):
        _k3_body(z_ref, s_ref, a_ref, b1_ref, w2_ref, b2_ref, out_ref)

    out = None
    for h in range(NSPLIT):
        def mk(off):
            return lambda i: (i + off, 0)
        zspec = pl.BlockSpec((bnk, D_LAT), mk(h * nb_h))
        sspec = pl.BlockSpec((bnk, D_LAT), lambda i: (i, 0))
        if h == 0:
            out = pl.pallas_call(
                _k3_body,
                in_specs=[zspec, sspec] + w_specs,
                out_specs=pl.BlockSpec((bnk, D_LAT), mk(0)),
                **k3_common,
            )(z_old, s_part[0], a128, b1pad, w2ext, b2pad)
        else:
            out = pl.pallas_call(
                _k3b_body,
                in_specs=[zspec, sspec] + w_specs
                + [pl.BlockSpec(memory_space=pltpu.HBM)],
                out_specs=pl.BlockSpec((bnk, D_LAT), mk(h * nb_h)),
                input_output_aliases={6: 0},
                **k3_common,
            )(z_old, s_part[h], a128, b1pad, w2ext, b2pad, out)
    return out


# speedup vs baseline: 1.0249x; 1.0249x over previous
import jax
import jax.numpy as jnp
from jax import lax
from jax.experimental import pallas as pl
from jax.experimental.pallas import tpu as pltpu
from jax.experimental.pallas import tpu_sc as plsc

N_P = 100000
D_LAT = 128
HIDDEN = 64
D_DYN = 32

NC = 2
NS = 16
NW = NC * NS

CH = 80
NCH = N_P // CH
RAW = 3 * CH
SUB = 80
NSUB = RAW // SUB
RAW_BUF = ((RAW + 15) // 16) * 16
K2MAX = ((NCH + NW - 1) // NW + 2 + 1) // 2


def _k1_body(z_ref, wab_ref, wc_ref, uu_ref):
    z = z_ref[...]
    uu_ref[0] = jnp.dot(z, wab_ref[...], preferred_element_type=jnp.float32)
    uu_ref[1] = jnp.dot(z, wc_ref[...], preferred_element_type=jnp.float32)


def _k3_body(z_ref, s_ref, a_ref, b1_ref, w2_ref, b2_ref, out_ref):
    z = z_ref[...]
    h = jnp.maximum(
        jnp.dot(z, a_ref[...], preferred_element_type=jnp.float32)
        + s_ref[...] + b1_ref[...], 0.0)
    out_ref[...] = z + jnp.dot(h, w2_ref[...],
                               preferred_element_type=jnp.float32) + b2_ref[...]


def _make_sc_body(T0, T1, K2):
  def _sc_body(uu_hbm, nlf_hbm, s_hbm,
               raw0, raw1, g0, g1, s0, s1, semg0, semg1, semw0, semw1):
    wid = lax.axis_index("s") * NC + lax.axis_index("c")
    iota = lax.iota(jnp.int32, 16)
    patt = [(((iota + r) % 3) >> 1) * N_P for r in range(3)]
    raws, gs, ss = (raw0, raw1), (g0, g1), (s0, s1)
    semgs, semws = (semg0, semg1), (semw0, semw1)

    def zrow(i, c):
        for j in range(4):
            s0[i, pl.ds(64 + 16 * j, 16)] = jnp.zeros((16,), jnp.float32)
            s1[i, pl.ds(64 + 16 * j, 16)] = jnp.zeros((16,), jnp.float32)
        return c

    lax.fori_loop(0, CH, zrow, 0)

    def prep_and_fire(b, t):
        pltpu.sync_copy(nlf_hbm.at[pl.ds(t * RAW, RAW)], raws[b].at[pl.ds(0, RAW)])
        for q in range(RAW_BUF // 16):
            sl = pl.ds(16 * q, 16)
            raws[b][sl] = raws[b][sl] + patt[q % 3]
        for j in range(NSUB):
            pltpu.async_copy(
                uu_hbm.at[raws[b].at[pl.ds(j * SUB, SUB)]],
                gs[b].at[pl.ds(j * SUB, SUB)],
                semgs[b],
            )

    def wait_gathers(b, t):
        for j in range(NSUB):
            pltpu.make_async_copy(
                uu_hbm.at[raws[b].at[pl.ds(j * SUB, SUB)]],
                gs[b].at[pl.ds(j * SUB, SUB)],
                semgs[b],
            ).wait()

    def step(m, b):
        t = T0 + wid + m * NW
        tn = t + NW
        tp = t - 2 * NW

        @pl.when(tn < T1)
        def _():
            prep_and_fire(b ^ 1, tn)

        @pl.when(jnp.logical_and(tp >= T0, tp < T1))
        def _():
            pltpu.make_async_copy(
                ss[b], s_hbm.at[pl.ds((tp - T0) * CH, CH)], semws[b]).wait()

        @pl.when(t < T1)
        def _():
            wait_gathers(b, t)

            def red(i, c):
                for j in range(4):
                    sl = pl.ds(16 * j, 16)
                    ss[b][i, sl] = (gs[b][3 * i, sl]
                                    + gs[b][3 * i + 1, pl.ds(64 + 16 * j, 16)]
                                    + gs[b][3 * i + 2, sl])
                return c

            lax.fori_loop(0, CH, red, 0)
            pltpu.async_copy(ss[b], s_hbm.at[pl.ds((t - T0) * CH, CH)], semws[b])

    @pl.when(T0 + wid < T1)
    def _():
        prep_and_fire(0, T0 + wid)

    def pair(k2, carry):
        step(2 * k2, 0)
        step(2 * k2 + 1, 1)
        return carry

    lax.fori_loop(0, K2, pair, 0)

  return _sc_body


def kernel(z_old, neighbour_list, W1, b1, W2, b2):
    n = z_old.shape[0]
    assert n == N_P
    nlf = neighbour_list.astype(jnp.int32).reshape(-1)

    w1r = W1.reshape(D_LAT, 4, HIDDEN)
    zero_h = jnp.zeros((D_LAT, HIDDEN), jnp.float32)
    wab = jnp.concatenate([w1r[:, 1, :], w1r[:, 2, :]], axis=1)
    wc = jnp.concatenate([w1r[:, 3, :], zero_h], axis=1)
    a128 = jnp.concatenate([w1r[:, 0, :], zero_h], axis=1)
    w2ext = jnp.zeros((D_LAT, D_LAT), jnp.float32).at[:HIDDEN, :D_DYN].set(W2)
    b1pad = jnp.zeros((1, D_LAT), jnp.float32).at[0, :HIDDEN].set(b1)
    b2pad = jnp.zeros((1, D_LAT), jnp.float32).at[0, :D_DYN].set(b2)

    bn = 2000
    grid1 = (n // bn,)
    uu3 = pl.pallas_call(
        _k1_body,
        grid=grid1,
        in_specs=[
            pl.BlockSpec((bn, D_LAT), lambda i: (i, 0)),
            pl.BlockSpec((D_LAT, D_LAT), lambda i: (0, 0)),
            pl.BlockSpec((D_LAT, D_LAT), lambda i: (0, 0)),
        ],
        out_specs=pl.BlockSpec((2, bn, D_LAT), lambda i: (0, i, 0)),
        out_shape=jax.ShapeDtypeStruct((2, n, D_LAT), jnp.float32),
    )(z_old, wab, wc)
    uu = uu3.reshape(2 * n, D_LAT)

    sc_mesh = plsc.VectorSubcoreMesh(core_axis_name="c", subcore_axis_name="s")
    sc_scratch = [
        pltpu.VMEM((RAW_BUF,), jnp.int32),
        pltpu.VMEM((RAW_BUF,), jnp.int32),
        pltpu.VMEM((RAW, D_LAT), jnp.float32),
        pltpu.VMEM((RAW, D_LAT), jnp.float32),
        pltpu.VMEM((CH, D_LAT), jnp.float32),
        pltpu.VMEM((CH, D_LAT), jnp.float32),
        pltpu.SemaphoreType.DMA,
        pltpu.SemaphoreType.DMA,
        pltpu.SemaphoreType.DMA,
        pltpu.SemaphoreType.DMA,
    ]
    NSPLIT = 5
    HCH = NCH // NSPLIT
    K2H = ((HCH + NW - 1) // NW + 2 + 1) // 2
    s_part = []
    for h in range(NSPLIT):
        s_part.append(pl.kernel(
            _make_sc_body(h * HCH, (h + 1) * HCH, K2H),
            out_type=jax.ShapeDtypeStruct((n // NSPLIT, D_LAT), jnp.float32),
            mesh=sc_mesh,
            scratch_types=sc_scratch,
            compiler_params=pltpu.CompilerParams(use_tc_tiling_on_sc=False),
        )(uu, nlf))

    bnk = 4000
    nb_h = n // NSPLIT // bnk
    k3_common = dict(
        grid=(nb_h,),
        out_shape=jax.ShapeDtypeStruct((n, D_LAT), jnp.float32),
    )
    w_specs = [
        pl.BlockSpec((D_LAT, D_LAT), lambda i: (0, 0)),
        pl.BlockSpec((1, D_LAT), lambda i: (0, 0)),
        pl.BlockSpec((D_LAT, D_LAT), lambda i: (0, 0)),
        pl.BlockSpec((1, D_LAT), lambda i: (0, 0)),
    ]

    def _k3b_body(z_ref, s_ref, a_ref, b1_ref, w2_ref, b2_ref, prev_ref, out_ref):
        _k3_body(z_ref, s_ref, a_ref, b1_ref, w2_ref, b2_ref, out_ref)

    out = None
    for h in range(NSPLIT):
        def mk(off):
            return lambda i: (i + off, 0)
        zspec = pl.BlockSpec((bnk, D_LAT), mk(h * nb_h))
        sspec = pl.BlockSpec((bnk, D_LAT), lambda i: (i, 0))
        if h == 0:
            out = pl.pallas_call(
                _k3_body,
                in_specs=[zspec, sspec] + w_specs,
                out_specs=pl.BlockSpec((bnk, D_LAT), mk(0)),
                **k3_common,
            )(z_old, s_part[0], a128, b1pad, w2ext, b2pad)
        else:
            out = pl.pallas_call(
                _k3b_body,
                in_specs=[zspec, sspec] + w_specs
                + [pl.BlockSpec(memory_space=pltpu.HBM)],
                out_specs=pl.BlockSpec((bnk, D_LAT), mk(h * nb_h)),
                input_output_aliases={6: 0},
                **k3_common,
            )(z_old, s_part[h], a128, b1pad, w2ext, b2pad, out)
    return out


# speedup vs baseline: 1.0713x; 1.0452x over previous
import jax
import jax.numpy as jnp
from jax import lax
from jax.experimental import pallas as pl
from jax.experimental.pallas import tpu as pltpu
from jax.experimental.pallas import tpu_sc as plsc

N_P = 100000
D_LAT = 128
HIDDEN = 64
D_DYN = 32

NC = 2
NS = 16
NW = NC * NS

CH = 80
NCH = N_P // CH
RAW = 3 * CH
SUB = 80
NSUB = RAW // SUB
RAW_BUF = ((RAW + 15) // 16) * 16
K2MAX = ((NCH + NW - 1) // NW + 2 + 1) // 2


def _k1_body(z_ref, wab_ref, wc_ref, uu_ref):
    z = z_ref[...]
    uu_ref[0] = jnp.dot(z, wab_ref[...], preferred_element_type=jnp.float32)
    uu_ref[1] = jnp.dot(z, wc_ref[...], preferred_element_type=jnp.float32)


def _k3_body(z_ref, s_ref, a_ref, b1_ref, w2_ref, b2_ref, out_ref):
    z = z_ref[...]
    h = jnp.maximum(
        jnp.dot(z, a_ref[...], preferred_element_type=jnp.float32)
        + s_ref[...] + b1_ref[...], 0.0)
    out_ref[...] = z + jnp.dot(h, w2_ref[...],
                               preferred_element_type=jnp.float32) + b2_ref[...]


def _make_sc_body(T0, T1, K2):
  def _sc_body(uu_hbm, nlf_hbm, s_hbm,
               raw0, raw1, g0, g1, s0, s1, semg0, semg1, semw0, semw1):
    wid = lax.axis_index("s") * NC + lax.axis_index("c")
    iota = lax.iota(jnp.int32, 16)
    patt = [(((iota + r) % 3) >> 1) * N_P for r in range(3)]
    raws, gs, ss = (raw0, raw1), (g0, g1), (s0, s1)
    semgs, semws = (semg0, semg1), (semw0, semw1)

    def zrow(i, c):
        for j in range(4):
            s0[i, pl.ds(64 + 16 * j, 16)] = jnp.zeros((16,), jnp.float32)
            s1[i, pl.ds(64 + 16 * j, 16)] = jnp.zeros((16,), jnp.float32)
        return c

    lax.fori_loop(0, CH, zrow, 0)

    def prep_and_fire(b, t):
        pltpu.sync_copy(nlf_hbm.at[pl.ds(t * RAW, RAW)], raws[b].at[pl.ds(0, RAW)])
        for q in range(RAW_BUF // 16):
            sl = pl.ds(16 * q, 16)
            raws[b][sl] = raws[b][sl] + patt[q % 3]
        for j in range(NSUB):
            pltpu.async_copy(
                uu_hbm.at[raws[b].at[pl.ds(j * SUB, SUB)]],
                gs[b].at[pl.ds(j * SUB, SUB)],
                semgs[b],
            )

    def wait_gathers(b, t):
        for j in range(NSUB):
            pltpu.make_async_copy(
                uu_hbm.at[raws[b].at[pl.ds(j * SUB, SUB)]],
                gs[b].at[pl.ds(j * SUB, SUB)],
                semgs[b],
            ).wait()

    def step(m, b):
        t = T0 + wid + m * NW
        tn = t + NW
        tp = t - 2 * NW

        @pl.when(tn < T1)
        def _():
            prep_and_fire(b ^ 1, tn)

        @pl.when(jnp.logical_and(tp >= T0, tp < T1))
        def _():
            pltpu.make_async_copy(
                ss[b], s_hbm.at[pl.ds((tp - T0) * CH, CH)], semws[b]).wait()

        @pl.when(t < T1)
        def _():
            wait_gathers(b, t)

            def red(i, c):
                for j in range(4):
                    sl = pl.ds(16 * j, 16)
                    ss[b][i, sl] = (gs[b][3 * i, sl]
                                    + gs[b][3 * i + 1, pl.ds(64 + 16 * j, 16)]
                                    + gs[b][3 * i + 2, sl])
                return c

            lax.fori_loop(0, CH, red, 0)
            pltpu.async_copy(ss[b], s_hbm.at[pl.ds((t - T0) * CH, CH)], semws[b])

    @pl.when(T0 + wid < T1)
    def _():
        prep_and_fire(0, T0 + wid)

    def pair(k2, carry):
        step(2 * k2, 0)
        step(2 * k2 + 1, 1)
        return carry

    lax.fori_loop(0, K2, pair, 0)

  return _sc_body


def kernel(z_old, neighbour_list, W1, b1, W2, b2):
    n = z_old.shape[0]
    assert n == N_P
    nlf = neighbour_list.astype(jnp.int32).reshape(-1)

    w1r = W1.reshape(D_LAT, 4, HIDDEN)
    zero_h = jnp.zeros((D_LAT, HIDDEN), jnp.float32)
    wab = jnp.concatenate([w1r[:, 1, :], w1r[:, 2, :]], axis=1)
    wc = jnp.concatenate([w1r[:, 3, :], zero_h], axis=1)
    a128 = jnp.concatenate([w1r[:, 0, :], zero_h], axis=1)
    w2ext = jnp.zeros((D_LAT, D_LAT), jnp.float32).at[:HIDDEN, :D_DYN].set(W2)
    b1pad = jnp.zeros((1, D_LAT), jnp.float32).at[0, :HIDDEN].set(b1)
    b2pad = jnp.zeros((1, D_LAT), jnp.float32).at[0, :D_DYN].set(b2)

    bn = 2000
    grid1 = (n // bn,)
    uu3 = pl.pallas_call(
        _k1_body,
        grid=grid1,
        in_specs=[
            pl.BlockSpec((bn, D_LAT), lambda i: (i, 0)),
            pl.BlockSpec((D_LAT, D_LAT), lambda i: (0, 0)),
            pl.BlockSpec((D_LAT, D_LAT), lambda i: (0, 0)),
        ],
        out_specs=pl.BlockSpec((2, bn, D_LAT), lambda i: (0, i, 0)),
        out_shape=jax.ShapeDtypeStruct((2, n, D_LAT), jnp.float32),
    )(z_old, wab, wc)
    uu = uu3.reshape(2 * n, D_LAT)

    sc_mesh = plsc.VectorSubcoreMesh(core_axis_name="c", subcore_axis_name="s")
    sc_scratch = [
        pltpu.VMEM((RAW_BUF,), jnp.int32),
        pltpu.VMEM((RAW_BUF,), jnp.int32),
        pltpu.VMEM((RAW, D_LAT), jnp.float32),
        pltpu.VMEM((RAW, D_LAT), jnp.float32),
        pltpu.VMEM((CH, D_LAT), jnp.float32),
        pltpu.VMEM((CH, D_LAT), jnp.float32),
        pltpu.SemaphoreType.DMA,
        pltpu.SemaphoreType.DMA,
        pltpu.SemaphoreType.DMA,
        pltpu.SemaphoreType.DMA,
    ]
    NSPLIT = 2
    HCH = NCH // NSPLIT
    K2H = ((HCH + NW - 1) // NW + 2 + 1) // 2
    s_part = []
    for h in range(NSPLIT):
        s_part.append(pl.kernel(
            _make_sc_body(h * HCH, (h + 1) * HCH, K2H),
            out_type=jax.ShapeDtypeStruct((n // NSPLIT, D_LAT), jnp.float32),
            mesh=sc_mesh,
            scratch_types=sc_scratch,
            compiler_params=pltpu.CompilerParams(use_tc_tiling_on_sc=False),
        )(uu, nlf))

    bnk = 5000
    nb_h = n // NSPLIT // bnk
    k3_common = dict(
        grid=(nb_h,),
        out_shape=jax.ShapeDtypeStruct((n, D_LAT), jnp.float32),
    )
    w_specs = [
        pl.BlockSpec((D_LAT, D_LAT), lambda i: (0, 0)),
        pl.BlockSpec((1, D_LAT), lambda i: (0, 0)),
        pl.BlockSpec((D_LAT, D_LAT), lambda i: (0, 0)),
        pl.BlockSpec((1, D_LAT), lambda i: (0, 0)),
    ]

    def _k3b_body(z_ref, s_ref, a_ref, b1_ref, w2_ref, b2_ref, prev_ref, out_ref):
        _k3_body(z_ref, s_ref, a_ref, b1_ref, w2_ref, b2_ref, out_ref)

    out = None
    for h in range(NSPLIT):
        def mk(off):
            return lambda i: (i + off, 0)
        zspec = pl.BlockSpec((bnk, D_LAT), mk(h * nb_h))
        sspec = pl.BlockSpec((bnk, D_LAT), lambda i: (i, 0))
        if h == 0:
            out = pl.pallas_call(
                _k3_body,
                in_specs=[zspec, sspec] + w_specs,
                out_specs=pl.BlockSpec((bnk, D_LAT), mk(0)),
                **k3_common,
            )(z_old, s_part[0], a128, b1pad, w2ext, b2pad)
        else:
            out = pl.pallas_call(
                _k3b_body,
                in_specs=[zspec, sspec] + w_specs
                + [pl.BlockSpec(memory_space=pltpu.HBM)],
                out_specs=pl.BlockSpec((bnk, D_LAT), mk(h * nb_h)),
                input_output_aliases={6: 0},
                **k3_common,
            )(z_old, s_part[h], a128, b1pad, w2ext, b2pad, out)
    return out
